# baseline (speedup 1.0000x reference)
import functools

import jax
import jax.numpy as jnp
from jax.experimental import pallas as pl
from jax.experimental.pallas import tpu as pltpu
from jax.experimental.pallas import tpu_sc as plsc

E = 16
TOP_K = 2
H = 1024
FF = 1024
ALPHA = 1.702
LIMIT = 7.0
NEG = -1e30


def _logits_kernel(x_ref, rw_ref, rb_ref, o_ref):
    o_ref[...] = jax.lax.dot_general(
        x_ref[...], rw_ref[...], (((1,), (1,)), ((), ())),
        preferred_element_type=jnp.float32) + rb_ref[...]


def _sc_shuf(vec, idx):
    dn = jax.lax.GatherDimensionNumbers(
        offset_dims=(), collapsed_slice_dims=(0,), start_index_map=(0,))
    return jax.lax.gather(
        vec, idx[:, None], dn, (1,),
        mode=jax.lax.GatherScatterMode.PROMISE_IN_BOUNDS)


def _sc_allmax(vec, iota):
    for k in (8, 4, 2, 1):
        vec = jnp.maximum(vec, _sc_shuf(vec, jnp.bitwise_xor(iota, k)))
    return vec


def _sc_allmin_i32(vec, iota):
    for k in (8, 4, 2, 1):
        vec = jnp.minimum(vec, _sc_shuf(vec, jnp.bitwise_xor(iota, k)))
    return vec


def _sc_router(logits_hbm, o_hbm, tmp_ref, sem):
    c = jax.lax.axis_index("c")
    s = jax.lax.axis_index("s")
    tec = c * 16 + s
    base = tec * 4
    pltpu.async_copy(logits_hbm.at[pl.ds(base, 4)], tmp_ref, sem).wait()
    iota = jax.lax.iota(jnp.int32, 16)
    for r in range(4):
        v = tmp_ref[r]
        m1 = _sc_allmax(v, iota)
        i1 = _sc_allmin_i32(jnp.where(v == m1, iota, E), iota)
        mask1 = iota == i1
        rest = jnp.where(mask1, NEG, v)
        m2 = _sc_allmax(rest, iota)
        i2 = _sc_allmin_i32(jnp.where(rest == m2, iota, E), iota)
        mask2 = iota == i2
        p1 = 1.0 / (1.0 + jnp.exp(m2 - m1))
        p2 = 1.0 - p1
        zero = jnp.zeros((16,), jnp.float32)
        tmp_ref[r] = (jnp.where(mask1, p1, zero)
                      + jnp.where(mask2, p2, zero))
    pltpu.async_copy(tmp_ref, o_hbm.at[pl.ds(base, 4)], sem).wait()


def _moe_kernel(x_ref, sc_ref, gwa_ref, gwb_ref, uwa_ref, uwb_ref,
                dwa_ref, dwb_ref, gb_ref, ub_ref, db_ref, out_ref):
    e = pl.program_id(0)
    xb = x_ref[...].astype(jnp.bfloat16)
    s = sc_ref[...]
    cols = jax.lax.broadcasted_iota(jnp.int32, s.shape, 1)
    contrib = None
    for i, (gw_ref, uw_ref, dw_ref) in enumerate(
            ((gwa_ref, uwa_ref, dwa_ref), (gwb_ref, uwb_ref, dwb_ref))):
        gate = jax.lax.dot_general(
            xb, gw_ref[0].astype(jnp.bfloat16), (((1,), (1,)), ((), ())),
            preferred_element_type=jnp.float32) + gb_ref[i]
        up = jax.lax.dot_general(
            xb, uw_ref[0].astype(jnp.bfloat16), (((1,), (1,)), ((), ())),
            preferred_element_type=jnp.float32) + ub_ref[i]
        gate = jnp.minimum(gate, LIMIT)
        up = jnp.clip(up, -LIMIT, LIMIT)
        glu = gate * jax.nn.sigmoid(gate * ALPHA)
        act = (up + 1.0) * glu
        y = jax.lax.dot_general(
            act.astype(jnp.bfloat16), dw_ref[0].astype(jnp.bfloat16),
            (((1,), (1,)), ((), ())),
            preferred_element_type=jnp.float32) + db_ref[i]
        w = jnp.sum(jnp.where(cols == 2 * e + i, s, 0.0),
                    axis=1, keepdims=True)
        contrib = w * y if contrib is None else contrib + w * y

    @pl.when(e == 0)
    def _init():
        out_ref[...] = contrib

    @pl.when(e != 0)
    def _acc():
        out_ref[...] += contrib


@functools.partial(jax.jit, static_argnums=())
def kernel(hidden_states, router_w, router_b, gate_w, gate_b, up_w, up_b,
           down_w, down_b):
    Bn, Tn, Hn = hidden_states.shape
    x = hidden_states.reshape(-1, Hn)
    Ttok = x.shape[0]
    rb2 = router_b.reshape(1, E)
    gb3 = gate_b.reshape(E, 1, FF)
    ub3 = up_b.reshape(E, 1, FF)
    db3 = down_b.reshape(E, 1, H)

    logits = pl.pallas_call(
        _logits_kernel,
        out_shape=jax.ShapeDtypeStruct((Ttok, E), jnp.float32),
    )(x, router_w, rb2)

    sc_fn = pl.kernel(
        _sc_router,
        out_type=jax.ShapeDtypeStruct((Ttok, E), jnp.float32),
        mesh=plsc.VectorSubcoreMesh(core_axis_name="c",
                                    subcore_axis_name="s"),
        scratch_types=[pltpu.VMEM((4, E), jnp.float32),
                       pltpu.SemaphoreType.DMA],
    )
    scores = sc_fn(logits)

    wspec_a = pl.BlockSpec((1, FF, H), lambda e: (2 * e, 0, 0))
    wspec_b = pl.BlockSpec((1, FF, H), lambda e: (2 * e + 1, 0, 0))

    (out,) = pl.pallas_call(
        _moe_kernel,
        grid=(E // 2,),
        in_specs=[
            pl.BlockSpec((Ttok, H), lambda e: (0, 0)),
            pl.BlockSpec((Ttok, E), lambda e: (0, 0)),
            wspec_a,
            wspec_b,
            wspec_a,
            wspec_b,
            wspec_a,
            wspec_b,
            pl.BlockSpec((2, 1, FF), lambda e: (e, 0, 0)),
            pl.BlockSpec((2, 1, FF), lambda e: (e, 0, 0)),
            pl.BlockSpec((2, 1, H), lambda e: (e, 0, 0)),
        ],
        out_specs=[
            pl.BlockSpec((Ttok, H), lambda e: (0, 0)),
        ],
        out_shape=[
            jax.ShapeDtypeStruct((Ttok, H), jnp.float32),
        ],
        compiler_params=pltpu.CompilerParams(
            dimension_semantics=("arbitrary",),
            vmem_limit_bytes=100 * 1024 * 1024,
        ),
    )(x, scores, gate_w, gate_w, up_w, up_w, down_w, down_w,
      gb3, ub3, db3)

    return out.reshape(Bn, Tn, Hn), scores


# speedup vs baseline: 1.2196x; 1.2196x over previous
import functools

import jax
import jax.numpy as jnp
from jax.experimental import pallas as pl
from jax.experimental.pallas import tpu as pltpu

E = 16
TOP_K = 2
H = 1024
FF = 1024
ALPHA = 1.702
LIMIT = 7.0
NEG = -1e30


def _moe_kernel(x_ref, rw_ref, rb_ref, gwa_ref, gwb_ref, uwa_ref, uwb_ref,
                dwa_ref, dwb_ref, gb_ref, ub_ref, db_ref, out_ref,
                scores_ref):
    e = pl.program_id(0)

    @pl.when(e == 0)
    def _router():
        x = x_ref[...]
        logits = jax.lax.dot_general(
            x, rw_ref[...], (((1,), (1,)), ((), ())),
            preferred_element_type=jnp.float32) + rb_ref[...]
        iota = jax.lax.broadcasted_iota(jnp.int32, logits.shape, 1)
        m1 = jnp.max(logits, axis=1, keepdims=True)
        idx1 = jnp.min(jnp.where(logits == m1, iota, E), axis=1, keepdims=True)
        mask1 = iota == idx1
        rest = jnp.where(mask1, NEG, logits)
        m2 = jnp.max(rest, axis=1, keepdims=True)
        idx2 = jnp.min(jnp.where(rest == m2, iota, E), axis=1, keepdims=True)
        mask2 = iota == idx2
        p1 = 1.0 / (1.0 + jnp.exp(m2 - m1))
        p2 = 1.0 - p1
        scores_ref[...] = jnp.where(mask1, p1, 0.0) + jnp.where(mask2, p2, 0.0)

    xb = x_ref[...].astype(jnp.bfloat16)
    s = scores_ref[...]
    cols = jax.lax.broadcasted_iota(jnp.int32, s.shape, 1)
    contrib = None
    for i, (gw_ref, uw_ref, dw_ref) in enumerate(
            ((gwa_ref, uwa_ref, dwa_ref), (gwb_ref, uwb_ref, dwb_ref))):
        gate = jax.lax.dot_general(
            xb, gw_ref[0].astype(jnp.bfloat16), (((1,), (1,)), ((), ())),
            preferred_element_type=jnp.float32) + gb_ref[i]
        up = jax.lax.dot_general(
            xb, uw_ref[0].astype(jnp.bfloat16), (((1,), (1,)), ((), ())),
            preferred_element_type=jnp.float32) + ub_ref[i]
        gate = jnp.minimum(gate, LIMIT)
        up = jnp.clip(up, -LIMIT, LIMIT)
        glu = gate * jax.nn.sigmoid(gate * ALPHA)
        act = (up + 1.0) * glu
        y = jax.lax.dot_general(
            act.astype(jnp.bfloat16), dw_ref[0].astype(jnp.bfloat16),
            (((1,), (1,)), ((), ())),
            preferred_element_type=jnp.float32) + db_ref[i]
        w = jnp.sum(jnp.where(cols == 2 * e + i, s, 0.0),
                    axis=1, keepdims=True)
        contrib = w * y if contrib is None else contrib + w * y

    @pl.when(e == 0)
    def _init():
        out_ref[...] = contrib

    @pl.when(e != 0)
    def _acc():
        out_ref[...] += contrib


@functools.partial(jax.jit, static_argnums=())
def kernel(hidden_states, router_w, router_b, gate_w, gate_b, up_w, up_b,
           down_w, down_b):
    Bn, Tn, Hn = hidden_states.shape
    x = hidden_states.reshape(-1, Hn)
    Ttok = x.shape[0]
    rb2 = router_b.reshape(1, E)
    gb3 = gate_b.reshape(E, 1, FF)
    ub3 = up_b.reshape(E, 1, FF)
    db3 = down_b.reshape(E, 1, H)

    wspec_a = pl.BlockSpec((1, FF, H), lambda e: (2 * e, 0, 0))
    wspec_b = pl.BlockSpec((1, FF, H), lambda e: (2 * e + 1, 0, 0))

    out, scores = pl.pallas_call(
        _moe_kernel,
        grid=(E // 2,),
        in_specs=[
            pl.BlockSpec((Ttok, H), lambda e: (0, 0)),
            pl.BlockSpec((E, H), lambda e: (0, 0)),
            pl.BlockSpec((1, E), lambda e: (0, 0)),
            wspec_a,
            wspec_b,
            wspec_a,
            wspec_b,
            wspec_a,
            wspec_b,
            pl.BlockSpec((2, 1, FF), lambda e: (e, 0, 0)),
            pl.BlockSpec((2, 1, FF), lambda e: (e, 0, 0)),
            pl.BlockSpec((2, 1, H), lambda e: (e, 0, 0)),
        ],
        out_specs=[
            pl.BlockSpec((Ttok, H), lambda e: (0, 0)),
            pl.BlockSpec((Ttok, E), lambda e: (0, 0)),
        ],
        out_shape=[
            jax.ShapeDtypeStruct((Ttok, H), jnp.float32),
            jax.ShapeDtypeStruct((Ttok, E), jnp.float32),
        ],
        compiler_params=pltpu.CompilerParams(
            dimension_semantics=("arbitrary",),
            vmem_limit_bytes=100 * 1024 * 1024,
        ),
    )(x, router_w, rb2, gate_w, gate_w, up_w, up_w, down_w, down_w,
      gb3, ub3, db3)

    return out.reshape(Bn, Tn, Hn), scores
